# baseline (speedup 1.0000x reference)
import functools

import jax
import jax.numpy as jnp
from jax.experimental import pallas as pl
from jax.experimental.pallas import tpu as pltpu

T, D, F, E, TOP_K = 2048, 1024, 1024, 8, 2
LIMIT = 7.0
ALPHA = 1.702
EPS = 1e-5

BP = 128
NPAD = 5120
NTILES = NPAD // BP


def _router_body(x_ref, nw_ref, rwt_ref, rb_ref,
                 t_ref, dest_ref, wp_ref, te_ref):
    xx = x_ref[...]
    ms = jnp.mean(xx * xx, axis=-1, keepdims=True)
    t = xx * jax.lax.rsqrt(ms + EPS) * nw_ref[...]
    t_ref[...] = t
    g = jnp.dot(t, rwt_ref[...], preferred_element_type=jnp.float32)
    g = g + rb_ref[...]
    m = jnp.max(g, axis=-1, keepdims=True)
    eg = jnp.exp(g - m)
    p = eg / jnp.sum(eg, axis=-1, keepdims=True)
    eidx = jax.lax.broadcasted_iota(jnp.int32, p.shape, 1)
    v1 = jnp.max(p, axis=-1, keepdims=True)
    i1 = jnp.min(jnp.where(p >= v1, eidx, E), axis=-1, keepdims=True)
    p2 = jnp.where(eidx == i1, -jnp.inf, p)
    v2 = jnp.max(p2, axis=-1, keepdims=True)
    i2 = jnp.min(jnp.where(p2 >= v2, eidx, E), axis=-1, keepdims=True)
    s = v1 + v2
    wp_ref[...] = jnp.concatenate([v1 / s, v2 / s], axis=1)

    oh = ((eidx == i1) | (eidx == i2)).astype(jnp.bfloat16)
    ri = jax.lax.broadcasted_iota(jnp.int32, (T, T), 0)
    ci = jax.lax.broadcasted_iota(jnp.int32, (T, T), 1)
    tri = (ri > ci).astype(jnp.bfloat16)
    acc = jnp.dot(tri, oh, preferred_element_type=jnp.float32)
    cnt = jnp.sum(oh.astype(jnp.float32), axis=0, keepdims=True)
    cp = jnp.ceil(cnt / BP) * BP
    ei8 = jax.lax.broadcasted_iota(jnp.int32, (E, E), 0)
    ej8 = jax.lax.broadcasted_iota(jnp.int32, (E, E), 1)
    upper = (ei8 < ej8).astype(jnp.float32)
    starts = jnp.dot(cp, upper, preferred_element_type=jnp.float32)
    total = jnp.sum(cp, axis=-1, keepdims=True)

    def sel(ik, mat):
        return jnp.sum(jnp.where(eidx == ik, mat, 0.0), axis=-1,
                       keepdims=True)

    starts_b = jnp.broadcast_to(starts, (T, E))
    d0 = sel(i1, starts_b) + sel(i1, acc)
    d1 = sel(i2, starts_b) + sel(i2, acc)
    dest_ref[...] = jnp.concatenate([d0, d1], axis=1).astype(jnp.int32)

    pos = (jax.lax.broadcasted_iota(jnp.int32, (64, E), 0) * BP).astype(
        jnp.float32)
    n_le = jnp.sum((jnp.broadcast_to(starts, (64, E)) <= pos).astype(
        jnp.int32), axis=-1, keepdims=True)
    tile_e = n_le - 1
    te_ref[...] = jnp.where(pos[:, :1] < total, tile_e, -1)


def _ffn_body(te_ref, g_ref, sw_ref, wg_ref, wu_ref, wd_ref,
              bg_ref, bu_ref, bd_ref, y_ref):
    ti = pl.program_id(0)

    @pl.when(te_ref[ti] >= 0)
    def _():
        t = g_ref[...].astype(jnp.bfloat16)
        gate = jnp.dot(t, wg_ref[0], preferred_element_type=jnp.float32)
        gate = gate + bg_ref[0]
        up = jnp.dot(t, wu_ref[0], preferred_element_type=jnp.float32)
        up = up + bu_ref[0]
        gate = jnp.minimum(gate, LIMIT)
        up = jnp.clip(up, -LIMIT, LIMIT)
        glu = gate * jax.nn.sigmoid(gate * ALPHA)
        act = ((up + 1.0) * glu).astype(jnp.bfloat16)
        y = jnp.dot(act, wd_ref[0], preferred_element_type=jnp.float32)
        y_ref[...] = (y + bd_ref[0]) * sw_ref[...]


@jax.jit
def _mlpblock(x, norm_w, router_w, router_b, w_gate_up, b_gate_up, w_down,
              b_down):
    rwt = router_w.T
    rb = router_b.reshape(1, E)
    wg = w_gate_up[:, :, 0::2].astype(jnp.bfloat16)
    wu = w_gate_up[:, :, 1::2].astype(jnp.bfloat16)
    bg = b_gate_up[:, 0::2].reshape(E, 1, F)
    bu = b_gate_up[:, 1::2].reshape(E, 1, F)
    wd = w_down.astype(jnp.bfloat16)
    bd = b_down.reshape(E, 1, D)
    nw = norm_w.reshape(1, D)

    t, dest, wp, te = pl.pallas_call(
        _router_body,
        grid=(1,),
        in_specs=[
            pl.BlockSpec((T, D), lambda i: (0, 0)),
            pl.BlockSpec((1, D), lambda i: (0, 0)),
            pl.BlockSpec((D, E), lambda i: (0, 0)),
            pl.BlockSpec((1, E), lambda i: (0, 0)),
        ],
        out_specs=[
            pl.BlockSpec((T, D), lambda i: (0, 0)),
            pl.BlockSpec((T, 2), lambda i: (0, 0)),
            pl.BlockSpec((T, 2), lambda i: (0, 0)),
            pl.BlockSpec((64, 1), lambda i: (0, 0)),
        ],
        out_shape=[
            jax.ShapeDtypeStruct((T, D), jnp.float32),
            jax.ShapeDtypeStruct((T, 2), jnp.int32),
            jax.ShapeDtypeStruct((T, 2), jnp.float32),
            jax.ShapeDtypeStruct((64, 1), jnp.int32),
        ],
    )(x, nw, rwt, rb)

    dest_flat = dest.reshape(2 * T)
    tid = jnp.arange(2 * T, dtype=jnp.int32) // 2
    sorted_tid = jnp.zeros((NPAD,), jnp.int32).at[dest_flat].set(tid)
    slot_w = jnp.zeros((NPAD,), jnp.float32).at[dest_flat].set(
        wp.reshape(2 * T))
    gathered = t[sorted_tid]

    te_flat = te.reshape(64)[:NTILES]
    y = pl.pallas_call(
        _ffn_body,
        grid_spec=pltpu.PrefetchScalarGridSpec(
            num_scalar_prefetch=1,
            grid=(NTILES,),
            in_specs=[
                pl.BlockSpec((BP, D), lambda ti, te: (ti, 0)),
                pl.BlockSpec((BP, 1), lambda ti, te: (ti, 0)),
                pl.BlockSpec((1, D, F),
                             lambda ti, te: (jnp.maximum(te[ti], 0), 0, 0)),
                pl.BlockSpec((1, D, F),
                             lambda ti, te: (jnp.maximum(te[ti], 0), 0, 0)),
                pl.BlockSpec((1, F, D),
                             lambda ti, te: (jnp.maximum(te[ti], 0), 0, 0)),
                pl.BlockSpec((1, 1, F),
                             lambda ti, te: (jnp.maximum(te[ti], 0), 0, 0)),
                pl.BlockSpec((1, 1, F),
                             lambda ti, te: (jnp.maximum(te[ti], 0), 0, 0)),
                pl.BlockSpec((1, 1, D),
                             lambda ti, te: (jnp.maximum(te[ti], 0), 0, 0)),
            ],
            out_specs=pl.BlockSpec((BP, D), lambda ti, te: (ti, 0)),
        ),
        out_shape=jax.ShapeDtypeStruct((NPAD, D), jnp.float32),
        compiler_params=pltpu.CompilerParams(
            dimension_semantics=("arbitrary",),
        ),
    )(te_flat, gathered, slot_w.reshape(NPAD, 1), wg, wu, wd, bg, bu, bd)

    out = x + y[dest[:, 0]] + y[dest[:, 1]]
    return out


def kernel(x, norm_w, router_w, router_b, w_gate_up, b_gate_up, w_down,
           b_down):
    return _mlpblock(x, norm_w, router_w, router_b, w_gate_up, b_gate_up,
                     w_down, b_down)


# speedup vs baseline: 1.0036x; 1.0036x over previous
import functools

import jax
import jax.numpy as jnp
from jax.experimental import pallas as pl
from jax.experimental.pallas import tpu as pltpu

T, D, F, E, TOP_K = 2048, 1024, 1024, 8, 2
LIMIT = 7.0
ALPHA = 1.702
EPS = 1e-5

BP = 128
NPAD = 5120
NTILES = NPAD // BP


def _router_body(x_ref, nw_ref, rwt_ref, rb_ref,
                 t_ref, dest_ref, wp_ref, te_ref):
    xx = x_ref[...]
    ms = jnp.mean(xx * xx, axis=-1, keepdims=True)
    t = xx * jax.lax.rsqrt(ms + EPS) * nw_ref[...]
    t_ref[...] = t
    g = jnp.dot(t, rwt_ref[...], preferred_element_type=jnp.float32)
    g = g + rb_ref[...]
    m = jnp.max(g, axis=-1, keepdims=True)
    eg = jnp.exp(g - m)
    p = eg / jnp.sum(eg, axis=-1, keepdims=True)
    eidx = jax.lax.broadcasted_iota(jnp.int32, p.shape, 1)
    v1 = jnp.max(p, axis=-1, keepdims=True)
    i1 = jnp.min(jnp.where(p >= v1, eidx, E), axis=-1, keepdims=True)
    p2 = jnp.where(eidx == i1, -jnp.inf, p)
    v2 = jnp.max(p2, axis=-1, keepdims=True)
    i2 = jnp.min(jnp.where(p2 >= v2, eidx, E), axis=-1, keepdims=True)
    s = v1 + v2
    wp_ref[...] = jnp.concatenate([v1 / s, v2 / s], axis=1)

    oh = ((eidx == i1) | (eidx == i2)).astype(jnp.bfloat16)
    ri = jax.lax.broadcasted_iota(jnp.int32, (T, T), 0)
    ci = jax.lax.broadcasted_iota(jnp.int32, (T, T), 1)
    tri = (ri > ci).astype(jnp.bfloat16)
    acc = jnp.dot(tri, oh, preferred_element_type=jnp.float32)
    cnt = jnp.sum(oh.astype(jnp.float32), axis=0, keepdims=True)
    cp = jnp.ceil(cnt / BP) * BP
    ei8 = jax.lax.broadcasted_iota(jnp.int32, (E, E), 0)
    ej8 = jax.lax.broadcasted_iota(jnp.int32, (E, E), 1)
    upper = (ei8 < ej8).astype(jnp.float32)
    starts = jnp.dot(cp, upper, preferred_element_type=jnp.float32)
    total = jnp.sum(cp, axis=-1, keepdims=True)

    def sel(ik, mat):
        return jnp.sum(jnp.where(eidx == ik, mat, 0.0), axis=-1,
                       keepdims=True)

    starts_b = jnp.broadcast_to(starts, (T, E))
    d0 = sel(i1, starts_b) + sel(i1, acc)
    d1 = sel(i2, starts_b) + sel(i2, acc)
    dest_ref[...] = jnp.concatenate([d0, d1], axis=1).astype(jnp.int32)

    pos = (jax.lax.broadcasted_iota(jnp.int32, (64, E), 0) * BP).astype(
        jnp.float32)
    n_le = jnp.sum((jnp.broadcast_to(starts, (64, E)) <= pos).astype(
        jnp.int32), axis=-1, keepdims=True)
    tile_e = n_le - 1
    te_ref[...] = jnp.where(pos[:, :1] < total, tile_e, -1)


def _ffn_body(te_ref, g_ref, sw_ref, wg_ref, wu_ref, wd_ref,
              bg_ref, bu_ref, bd_ref, y_ref):
    ti = pl.program_id(0)
    e = jnp.maximum(te_ref[ti], 0)

    @pl.when(te_ref[ti] >= 0)
    def _():
        t = g_ref[...].astype(jnp.bfloat16)
        gate = jnp.dot(t, wg_ref[e], preferred_element_type=jnp.float32)
        gate = gate + bg_ref[e]
        up = jnp.dot(t, wu_ref[e], preferred_element_type=jnp.float32)
        up = up + bu_ref[e]
        gate = jnp.minimum(gate, LIMIT)
        up = jnp.clip(up, -LIMIT, LIMIT)
        glu = gate * jax.nn.sigmoid(gate * ALPHA)
        act = ((up + 1.0) * glu).astype(jnp.bfloat16)
        y = jnp.dot(act, wd_ref[e], preferred_element_type=jnp.float32)
        y_ref[...] = (y + bd_ref[e]) * sw_ref[...]


@jax.jit
def _mlpblock(x, norm_w, router_w, router_b, w_gate_up, b_gate_up, w_down,
              b_down):
    rwt = router_w.T
    rb = router_b.reshape(1, E)
    wg = w_gate_up[:, :, 0::2].astype(jnp.bfloat16)
    wu = w_gate_up[:, :, 1::2].astype(jnp.bfloat16)
    bg = b_gate_up[:, 0::2].reshape(E, 1, F)
    bu = b_gate_up[:, 1::2].reshape(E, 1, F)
    wd = w_down.astype(jnp.bfloat16)
    bd = b_down.reshape(E, 1, D)
    nw = norm_w.reshape(1, D)

    t, dest, wp, te = pl.pallas_call(
        _router_body,
        grid=(1,),
        in_specs=[
            pl.BlockSpec((T, D), lambda i: (0, 0)),
            pl.BlockSpec((1, D), lambda i: (0, 0)),
            pl.BlockSpec((D, E), lambda i: (0, 0)),
            pl.BlockSpec((1, E), lambda i: (0, 0)),
        ],
        out_specs=[
            pl.BlockSpec((T, D), lambda i: (0, 0)),
            pl.BlockSpec((T, 2), lambda i: (0, 0)),
            pl.BlockSpec((T, 2), lambda i: (0, 0)),
            pl.BlockSpec((64, 1), lambda i: (0, 0)),
        ],
        out_shape=[
            jax.ShapeDtypeStruct((T, D), jnp.float32),
            jax.ShapeDtypeStruct((T, 2), jnp.int32),
            jax.ShapeDtypeStruct((T, 2), jnp.float32),
            jax.ShapeDtypeStruct((64, 1), jnp.int32),
        ],
    )(x, nw, rwt, rb)

    dest_flat = dest.reshape(2 * T)
    tid = jnp.arange(2 * T, dtype=jnp.int32) // 2
    sorted_tid = jnp.zeros((NPAD,), jnp.int32).at[dest_flat].set(tid)
    slot_w = jnp.zeros((NPAD,), jnp.float32).at[dest_flat].set(
        wp.reshape(2 * T))
    gathered = t[sorted_tid]

    te_flat = te.reshape(64)[:NTILES]
    y = pl.pallas_call(
        _ffn_body,
        grid_spec=pltpu.PrefetchScalarGridSpec(
            num_scalar_prefetch=1,
            grid=(NTILES,),
            in_specs=[
                pl.BlockSpec((BP, D), lambda ti, te: (ti, 0)),
                pl.BlockSpec((BP, 1), lambda ti, te: (ti, 0)),
                pl.BlockSpec((E, D, F), lambda ti, te: (0, 0, 0)),
                pl.BlockSpec((E, D, F), lambda ti, te: (0, 0, 0)),
                pl.BlockSpec((E, F, D), lambda ti, te: (0, 0, 0)),
                pl.BlockSpec((E, 1, F), lambda ti, te: (0, 0, 0)),
                pl.BlockSpec((E, 1, F), lambda ti, te: (0, 0, 0)),
                pl.BlockSpec((E, 1, D), lambda ti, te: (0, 0, 0)),
            ],
            out_specs=pl.BlockSpec((BP, D), lambda ti, te: (ti, 0)),
        ),
        out_shape=jax.ShapeDtypeStruct((NPAD, D), jnp.float32),
        compiler_params=pltpu.CompilerParams(
            dimension_semantics=("arbitrary",),
            vmem_limit_bytes=120 * 1024 * 1024,
        ),
    )(te_flat, gathered, slot_w.reshape(NPAD, 1), wg, wu, wd, bg, bu, bd)

    out = x + y[dest[:, 0]] + y[dest[:, 1]]
    return out


def kernel(x, norm_w, router_w, router_b, w_gate_up, b_gate_up, w_down,
           b_down):
    return _mlpblock(x, norm_w, router_w, router_b, w_gate_up, b_gate_up,
                     w_down, b_down)


# speedup vs baseline: 3.4049x; 3.3927x over previous
import functools

import jax
import jax.numpy as jnp
from jax.experimental import pallas as pl
from jax.experimental.pallas import tpu as pltpu

T, D, F, E, TOP_K = 2048, 1024, 1024, 8, 2
LIMIT = 7.0
ALPHA = 1.702
EPS = 1e-5

BP = 128
NPAD = 5120
NTILES = NPAD // BP


def _deint_body(p_ref, wg_ref, wu_ref):
    v = p_ref[0]
    wg_ref[0] = jax.lax.bitcast_convert_type(
        v << 16, jnp.float32).astype(jnp.bfloat16)
    wu_ref[0] = jax.lax.bitcast_convert_type(
        v & jnp.int32(-65536), jnp.float32).astype(jnp.bfloat16)


def _router_body(x_ref, nw_ref, rwt_ref, rb_ref,
                 t_ref, dest_ref, wp_ref, te_ref):
    xx = x_ref[...]
    ms = jnp.mean(xx * xx, axis=-1, keepdims=True)
    t = xx * jax.lax.rsqrt(ms + EPS) * nw_ref[...]
    t_ref[...] = t
    g = jnp.dot(t, rwt_ref[...], preferred_element_type=jnp.float32)
    g = g + rb_ref[...]
    m = jnp.max(g, axis=-1, keepdims=True)
    eg = jnp.exp(g - m)
    p = eg / jnp.sum(eg, axis=-1, keepdims=True)
    eidx = jax.lax.broadcasted_iota(jnp.int32, p.shape, 1)
    v1 = jnp.max(p, axis=-1, keepdims=True)
    i1 = jnp.min(jnp.where(p >= v1, eidx, E), axis=-1, keepdims=True)
    p2 = jnp.where(eidx == i1, -jnp.inf, p)
    v2 = jnp.max(p2, axis=-1, keepdims=True)
    i2 = jnp.min(jnp.where(p2 >= v2, eidx, E), axis=-1, keepdims=True)
    s = v1 + v2
    wp_ref[...] = jnp.concatenate([v1 / s, v2 / s], axis=1)

    oh = ((eidx == i1) | (eidx == i2)).astype(jnp.bfloat16)
    ri = jax.lax.broadcasted_iota(jnp.int32, (T, T), 0)
    ci = jax.lax.broadcasted_iota(jnp.int32, (T, T), 1)
    tri = (ri > ci).astype(jnp.bfloat16)
    acc = jnp.dot(tri, oh, preferred_element_type=jnp.float32)
    cnt = jnp.sum(oh.astype(jnp.float32), axis=0, keepdims=True)
    cp = jnp.ceil(cnt / BP) * BP
    ei8 = jax.lax.broadcasted_iota(jnp.int32, (E, E), 0)
    ej8 = jax.lax.broadcasted_iota(jnp.int32, (E, E), 1)
    upper = (ei8 < ej8).astype(jnp.float32)
    starts = jnp.dot(cp, upper, preferred_element_type=jnp.float32)
    total = jnp.sum(cp, axis=-1, keepdims=True)

    def sel(ik, mat):
        return jnp.sum(jnp.where(eidx == ik, mat, 0.0), axis=-1,
                       keepdims=True)

    starts_b = jnp.broadcast_to(starts, (T, E))
    d0 = sel(i1, starts_b) + sel(i1, acc)
    d1 = sel(i2, starts_b) + sel(i2, acc)
    dest_ref[...] = jnp.concatenate([d0, d1], axis=1).astype(jnp.int32)

    pos = (jax.lax.broadcasted_iota(jnp.int32, (64, E), 0) * BP).astype(
        jnp.float32)
    n_le = jnp.sum((jnp.broadcast_to(starts, (64, E)) <= pos).astype(
        jnp.int32), axis=-1, keepdims=True)
    tile_e = n_le - 1
    te_ref[...] = jnp.where(pos[:, :1] < total, tile_e, -1)


def _ffn_body(te_ref, g_ref, sw_ref, wg_ref, wu_ref, wd_ref,
              bg_ref, bu_ref, bd_ref, y_ref):
    ti = pl.program_id(0)
    e = jnp.maximum(te_ref[ti], 0)

    @pl.when(te_ref[ti] >= 0)
    def _():
        t = g_ref[...].astype(jnp.bfloat16)
        gate = jnp.dot(t, wg_ref[e], preferred_element_type=jnp.float32)
        gate = gate + bg_ref[e]
        up = jnp.dot(t, wu_ref[e], preferred_element_type=jnp.float32)
        up = up + bu_ref[e]
        gate = jnp.minimum(gate, LIMIT)
        up = jnp.clip(up, -LIMIT, LIMIT)
        glu = gate * jax.nn.sigmoid(gate * ALPHA)
        act = ((up + 1.0) * glu).astype(jnp.bfloat16)
        y = jnp.dot(act, wd_ref[e], preferred_element_type=jnp.float32)
        y_ref[...] = (y + bd_ref[e]) * sw_ref[...]


@jax.jit
def _mlpblock(x, norm_w, router_w, router_b, w_gate_up, b_gate_up, w_down,
              b_down):
    rwt = router_w.T
    rb = router_b.reshape(1, E)
    bg = b_gate_up[:, 0::2].reshape(E, 1, F)
    bu = b_gate_up[:, 1::2].reshape(E, 1, F)
    wd = w_down.astype(jnp.bfloat16)
    bd = b_down.reshape(E, 1, D)
    nw = norm_w.reshape(1, D)

    wgu_i32 = jax.lax.bitcast_convert_type(
        w_gate_up.astype(jnp.bfloat16).reshape(E, D, F, 2), jnp.int32)
    wg, wu = pl.pallas_call(
        _deint_body,
        grid=(E,),
        in_specs=[pl.BlockSpec((1, D, F), lambda e_: (e_, 0, 0))],
        out_specs=[pl.BlockSpec((1, D, F), lambda e_: (e_, 0, 0)),
                   pl.BlockSpec((1, D, F), lambda e_: (e_, 0, 0))],
        out_shape=[jax.ShapeDtypeStruct((E, D, F), jnp.bfloat16),
                   jax.ShapeDtypeStruct((E, D, F), jnp.bfloat16)],
    )(wgu_i32)

    t, dest, wp, te = pl.pallas_call(
        _router_body,
        grid=(1,),
        in_specs=[
            pl.BlockSpec((T, D), lambda i: (0, 0)),
            pl.BlockSpec((1, D), lambda i: (0, 0)),
            pl.BlockSpec((D, E), lambda i: (0, 0)),
            pl.BlockSpec((1, E), lambda i: (0, 0)),
        ],
        out_specs=[
            pl.BlockSpec((T, D), lambda i: (0, 0)),
            pl.BlockSpec((T, 2), lambda i: (0, 0)),
            pl.BlockSpec((T, 2), lambda i: (0, 0)),
            pl.BlockSpec((64, 1), lambda i: (0, 0)),
        ],
        out_shape=[
            jax.ShapeDtypeStruct((T, D), jnp.float32),
            jax.ShapeDtypeStruct((T, 2), jnp.int32),
            jax.ShapeDtypeStruct((T, 2), jnp.float32),
            jax.ShapeDtypeStruct((64, 1), jnp.int32),
        ],
    )(x, nw, rwt, rb)

    dest_flat = dest.reshape(2 * T)
    tid = jnp.arange(2 * T, dtype=jnp.int32) // 2
    sorted_tid = jnp.zeros((NPAD,), jnp.int32).at[dest_flat].set(tid)
    slot_w = jnp.zeros((NPAD,), jnp.float32).at[dest_flat].set(
        wp.reshape(2 * T))
    gathered = t[sorted_tid]

    te_flat = te.reshape(64)[:NTILES]
    y = pl.pallas_call(
        _ffn_body,
        grid_spec=pltpu.PrefetchScalarGridSpec(
            num_scalar_prefetch=1,
            grid=(NTILES,),
            in_specs=[
                pl.BlockSpec((BP, D), lambda ti, te: (ti, 0)),
                pl.BlockSpec((BP, 1), lambda ti, te: (ti, 0)),
                pl.BlockSpec((E, D, F), lambda ti, te: (0, 0, 0)),
                pl.BlockSpec((E, D, F), lambda ti, te: (0, 0, 0)),
                pl.BlockSpec((E, F, D), lambda ti, te: (0, 0, 0)),
                pl.BlockSpec((E, 1, F), lambda ti, te: (0, 0, 0)),
                pl.BlockSpec((E, 1, F), lambda ti, te: (0, 0, 0)),
                pl.BlockSpec((E, 1, D), lambda ti, te: (0, 0, 0)),
            ],
            out_specs=pl.BlockSpec((BP, D), lambda ti, te: (ti, 0)),
        ),
        out_shape=jax.ShapeDtypeStruct((NPAD, D), jnp.float32),
        compiler_params=pltpu.CompilerParams(
            dimension_semantics=("arbitrary",),
            vmem_limit_bytes=120 * 1024 * 1024,
        ),
    )(te_flat, gathered, slot_w.reshape(NPAD, 1), wg, wu, wd, bg, bu, bd)

    out = x + y[dest[:, 0]] + y[dest[:, 1]]
    return out


def kernel(x, norm_w, router_w, router_b, w_gate_up, b_gate_up, w_down,
           b_down):
    return _mlpblock(x, norm_w, router_w, router_b, w_gate_up, b_gate_up,
                     w_down, b_down)
